# Initial kernel scaffold; baseline (speedup 1.0000x reference)
import jax
import jax.numpy as jnp
from jax.experimental import pallas as pl


def kernel(query, enity_info, k):
    raise NotImplementedError("write your pallas kernel here")


# speedup vs baseline: 1.7006x; 1.7006x over previous
import jax
import jax.numpy as jnp
from jax import lax
from jax.experimental import pallas as pl
from jax.experimental.pallas import tpu as pltpu
from jax.experimental.pallas import tpu_sc as plsc

Q = 64
D = 256
E = 4096
TOPK = 10
NC = 2
NS = 16
NW = NC * NS
L = 16
EPW = E // NW
G = EPW // L


def _sc_body(q_hbm, et_hbm, scores_hbm, q_v, et_v, jv_v):
    c = lax.axis_index("c")
    s = lax.axis_index("s")
    wid = s * NC + c
    pltpu.sync_copy(q_hbm, q_v)
    pltpu.sync_copy(et_hbm.at[wid], et_v)

    def se_body(d, accs):
        return tuple(accs[g] + et_v[d, pl.ds(g * L, L)] for g in range(G))

    se = lax.fori_loop(
        0, D, se_body, tuple(jnp.zeros((L,), jnp.float32) for _ in range(G))
    )

    def q_body(qi, _):
        def dc_body(dc, carry):
            accs, sq = list(carry[:G]), carry[G]
            qv = q_v[qi, pl.ds(dc * L, L)]
            d0 = dc * L
            for j in range(L):
                qb = lax.broadcast(qv[j], (L,))
                for g in range(G):
                    accs[g] = accs[g] + jnp.minimum(
                        et_v[d0 + j, pl.ds(g * L, L)], qb
                    )
                sq = sq + qb
            return tuple(accs) + (sq,)

        carry = lax.fori_loop(
            0, D // L, dc_body,
            tuple(jnp.zeros((L,), jnp.float32) for _ in range(G + 1)),
        )
        accs, sq = carry[:G], carry[G]
        for g in range(G):
            jv_v[qi, pl.ds(g * L, L)] = accs[g] / (sq + se[g] - accs[g])
        return 0

    lax.fori_loop(0, Q, q_body, 0)

    pltpu.sync_copy(jv_v, scores_hbm.at[:, pl.ds(wid * EPW, EPW)])


def _tc_topk_body(s_ref, top_ref, bot_ref):
    big = jnp.int32(2**30)
    neg = jnp.float32(-3e38)
    ent_iota = lax.broadcasted_iota(jnp.int32, (Q, E), 1)

    def select10(cur):
        outs = []
        for _ in range(TOPK):
            m = jnp.max(cur, axis=1, keepdims=True)
            hit = cur == m
            ent = jnp.min(jnp.where(hit, ent_iota, big), axis=1,
                          keepdims=True)
            outs.append(ent)
            cur = jnp.where(ent == ent_iota, neg, cur)
        return jnp.concatenate(outs, axis=1)

    scores = s_ref[...]
    top_ref[...] = select10(scores)
    bot_ref[...] = select10(-scores)


def kernel(query, enity_info, k):
    q2 = query.reshape(Q, D)
    eb = enity_info.reshape(NW, EPW, D).transpose(0, 2, 1)

    sc = pl.kernel(
        _sc_body,
        out_type=[jax.ShapeDtypeStruct((Q, E), jnp.float32)],
        mesh=plsc.VectorSubcoreMesh(
            core_axis_name="c", subcore_axis_name="s",
            num_cores=NC, num_subcores=NS,
        ),
        scratch_types=[
            pltpu.VMEM((Q, D), jnp.float32),
            pltpu.VMEM((D, EPW), jnp.float32),
            pltpu.VMEM((Q, EPW), jnp.float32),
        ],
    )
    scores, = sc(q2, eb)

    top, bot = pl.pallas_call(
        _tc_topk_body,
        out_shape=[
            jax.ShapeDtypeStruct((Q, TOPK), jnp.int32),
            jax.ShapeDtypeStruct((Q, TOPK), jnp.int32),
        ],
    )(scores)

    kd = jnp.asarray(k - TOPK, jnp.int32)
    return top + kd, bot + kd
